# baseline (speedup 1.0000x reference)
import functools

import jax
import jax.numpy as jnp
from jax import lax
from jax.experimental import pallas as pl
from jax.experimental.pallas import tpu as pltpu
from jax.experimental.pallas import tpu_sc as plsc

T = 2048
D = 1024
F = 4096
E = 8
BM = 128
NB = T * 2 // BM + E
NPAD = NB * BM
NW = 32
TPW = T // NW
CH = 32
CHUNK = 256


def _router_body(x_ref, rw_ref, p0_ref, p1_ref, g0_ref, g1_ref, ba_ref):
    x = x_ref[...]
    rw = rw_ref[...]
    logits = lax.dot_general(x, rw, (((1,), (1,)), ((), ())),
                             preferred_element_type=jnp.float32)
    iota = lax.broadcasted_iota(jnp.int32, logits.shape, 1)
    m1 = jnp.max(logits, axis=1, keepdims=True)
    i1 = jnp.min(jnp.where(logits == m1, iota, E), axis=1, keepdims=True)
    oh0 = (iota == i1)
    l2 = jnp.where(oh0, -jnp.inf, logits)
    m2 = jnp.max(l2, axis=1, keepdims=True)
    i2 = jnp.min(jnp.where(l2 == m2, iota, E), axis=1, keepdims=True)
    oh1 = (iota == i2)
    g1 = 1.0 / (1.0 + jnp.exp(m2 - m1))
    g0_ref[...] = jnp.broadcast_to(g1, (T, 16))
    g1_ref[...] = jnp.broadcast_to(1.0 - g1, (T, 16))

    oh0f = oh0.astype(jnp.float32)
    oh1f = oh1.astype(jnp.float32)
    oh = oh0f + oh1f

    r = lax.broadcasted_iota(jnp.int32, (CHUNK, CHUNK), 0)
    c = lax.broadcasted_iota(jnp.int32, (CHUNK, CHUNK), 1)
    tri = (r >= c).astype(jnp.float32)
    nch = T // CHUNK
    incs = []
    for ci in range(nch):
        oh_c = lax.slice(oh, (ci * CHUNK, 0), ((ci + 1) * CHUNK, E))
        incs.append(lax.dot_general(tri, oh_c, (((1,), (0,)), ((), ())),
                                    preferred_element_type=jnp.float32))
    tot = jnp.concatenate(
        [lax.slice(inc, (CHUNK - 1, 0), (CHUNK, E)) for inc in incs], axis=0)
    r8 = lax.broadcasted_iota(jnp.int32, (nch, nch), 0)
    c8 = lax.broadcasted_iota(jnp.int32, (nch, nch), 1)
    strict8 = (r8 < c8).astype(jnp.float32)
    chunk_excl = lax.dot_general(strict8, tot, (((0,), (0,)), ((), ())),
                                 preferred_element_type=jnp.float32)
    counts = jnp.sum(tot, axis=0, keepdims=True)

    nblk = jnp.floor((counts + (BM - 1)) * (1.0 / BM))
    rE = lax.broadcasted_iota(jnp.int32, (E, E), 0)
    cE = lax.broadcasted_iota(jnp.int32, (E, E), 1)
    strictE = (rE < cE).astype(jnp.float32)
    blk_off = lax.dot_general(nblk, strictE, (((1,), (0,)), ((), ())),
                              preferred_element_type=jnp.float32)
    padded_off = blk_off * BM

    for ci in range(nch):
        oh_c = lax.slice(oh, (ci * CHUNK, 0), ((ci + 1) * CHUNK, E))
        oh0_c = lax.slice(oh0f, (ci * CHUNK, 0), ((ci + 1) * CHUNK, E))
        oh1_c = lax.slice(oh1f, (ci * CHUNK, 0), ((ci + 1) * CHUNK, E))
        excl_c = lax.slice(chunk_excl, (ci, 0), (ci + 1, E))
        before = incs[ci] - oh_c + excl_c + padded_off
        p0 = jnp.sum(before * oh0_c, axis=1, keepdims=True)
        p1 = jnp.sum((before + oh0_c) * oh1_c, axis=1, keepdims=True)
        sl = pl.ds(ci * CHUNK, CHUNK)
        p0_ref[sl, :] = p0.astype(jnp.int32)
        p1_ref[sl, :] = p1.astype(jnp.int32)

    blk_offT = lax.dot_general(jnp.eye(E, dtype=jnp.float32), blk_off,
                               (((1,), (1,)), ((), ())),
                               preferred_element_type=jnp.float32)
    biota = lax.broadcasted_iota(jnp.int32, (E, NB), 1).astype(jnp.float32)
    ge = (biota >= blk_offT).astype(jnp.float32)
    be = jnp.sum(ge, axis=0, keepdims=True) - 1.0
    total = jnp.sum(nblk, axis=1, keepdims=True)
    b1 = lax.broadcasted_iota(jnp.int32, (1, NB), 1).astype(jnp.float32)
    act = (b1 < total).astype(jnp.float32)
    ba_ref[...] = jnp.concatenate([be, act], axis=0).astype(jnp.int32)


def _router(flat, router_w):
    return pl.pallas_call(
        _router_body,
        out_shape=(jax.ShapeDtypeStruct((T, 1), jnp.int32),
                   jax.ShapeDtypeStruct((T, 1), jnp.int32),
                   jax.ShapeDtypeStruct((T, 16), jnp.float32),
                   jax.ShapeDtypeStruct((T, 16), jnp.float32),
                   jax.ShapeDtypeStruct((2, NB), jnp.int32)),
    )(flat, router_w)


def _sc_mesh():
    return plsc.VectorSubcoreMesh(core_axis_name="c", subcore_axis_name="s")


def _dispatch(flat, pos0, pos1):
    @functools.partial(
        pl.kernel,
        mesh=_sc_mesh(),
        out_type=jax.ShapeDtypeStruct((NPAD, D), jnp.float32),
        scratch_types=[
            pltpu.VMEM((TPW,), jnp.int32),
            pltpu.VMEM((TPW,), jnp.int32),
            pltpu.VMEM((TPW, D), jnp.float32),
            pltpu.SemaphoreType.DMA,
        ],
    )
    def k(flat_hbm, p0_hbm, p1_hbm, xg_hbm, i0_v, i1_v, rows_v, sem):
        wid = lax.axis_index("s") * 2 + lax.axis_index("c")
        base = wid * TPW
        sl = pl.ds(base, TPW)
        pltpu.sync_copy(p0_hbm.at[sl], i0_v)
        pltpu.sync_copy(p1_hbm.at[sl], i1_v)
        pltpu.sync_copy(flat_hbm.at[sl], rows_v)
        a = pltpu.async_copy(rows_v, xg_hbm.at[i0_v], sem)
        b = pltpu.async_copy(rows_v, xg_hbm.at[i1_v], sem)
        a.wait()
        b.wait()

    return k(flat, pos0, pos1)


BF = 1024
NF = F // BF


def _moe_body(ba_ref, xg_ref, w1_ref, w3_ref, w2_ref, y_ref, acc_ref):
    f = pl.program_id(0)
    b = pl.program_id(1)
    active = ba_ref[1, b] == 1
    sl = pl.ds(b * BM, BM)

    @pl.when(active)
    def _():
        xb = xg_ref[...]
        a = lax.dot_general(xb, w1_ref[0], (((1,), (1,)), ((), ())),
                            preferred_element_type=jnp.float32)
        c = lax.dot_general(xb, w3_ref[0], (((1,), (1,)), ((), ())),
                            preferred_element_type=jnp.float32)
        h = (a * jax.nn.sigmoid(a)) * c
        p = lax.dot_general(h, w2_ref[0], (((1,), (1,)), ((), ())),
                            preferred_element_type=jnp.float32)

        @pl.when(f == 0)
        def _():
            acc_ref[sl, :] = p

        @pl.when(f > 0)
        def _():
            acc_ref[sl, :] += p

    @pl.when(f == NF - 1)
    def _():
        y_ref[...] = acc_ref[sl, :]


def _moe_outer(ba_ref, xg_hbm, w1_hbm, w3_hbm, w2_hbm, y_hbm, acc_ref):
    def inner(xg_ref, w1_ref, w3_ref, w2_ref, y_ref):
        _moe_body(ba_ref, xg_ref, w1_ref, w3_ref, w2_ref, y_ref, acc_ref)

    look = pl.Buffered(buffer_count=3, use_lookahead=True)
    pipe = pltpu.emit_pipeline(
        inner,
        grid=(NF, NB),
        in_specs=[
            pl.BlockSpec((BM, D),
                         lambda f, b: (jnp.where(ba_ref[1, b] == 1, b, 0), 0)),
            pl.BlockSpec((1, BF, D), lambda f, b: (ba_ref[0, b], f, 0),
                         pipeline_mode=look),
            pl.BlockSpec((1, BF, D), lambda f, b: (ba_ref[0, b], f, 0),
                         pipeline_mode=look),
            pl.BlockSpec((1, D, BF), lambda f, b: (ba_ref[0, b], 0, f),
                         pipeline_mode=pl.Buffered(buffer_count=2,
                                                   use_lookahead=True)),
        ],
        out_specs=[pl.BlockSpec(
            (BM, D), lambda f, b: (jnp.where(f == NF - 1, b, 0), 0))],
    )
    pipe(xg_hbm, w1_hbm, w3_hbm, w2_hbm, y_hbm)


def _moe(be_act, xg, w1, w3, w2):
    return pl.pallas_call(
        _moe_outer,
        in_specs=[
            pl.BlockSpec(memory_space=pltpu.SMEM),
            pl.BlockSpec(memory_space=pl.ANY),
            pl.BlockSpec(memory_space=pl.ANY),
            pl.BlockSpec(memory_space=pl.ANY),
            pl.BlockSpec(memory_space=pl.ANY),
        ],
        out_specs=pl.BlockSpec(memory_space=pl.ANY),
        scratch_shapes=[pltpu.VMEM((NPAD, D), jnp.float32)],
        out_shape=jax.ShapeDtypeStruct((NPAD, D), jnp.float32),
        compiler_params=pltpu.CompilerParams(
            vmem_limit_bytes=128 * 1024 * 1024),
    )(be_act, xg, w1, w3, w2)


def _combine(y, pos0, pos1, g0, g1):
    @functools.partial(
        pl.kernel,
        mesh=_sc_mesh(),
        out_type=jax.ShapeDtypeStruct((T, D), jnp.float32),
        scratch_types=[
            pltpu.VMEM((CH,), jnp.int32),
            pltpu.VMEM((CH,), jnp.int32),
            pltpu.VMEM((CH, 16), jnp.float32),
            pltpu.VMEM((CH, 16), jnp.float32),
            pltpu.VMEM((CH, D), jnp.float32),
            pltpu.VMEM((CH, D), jnp.float32),
            pltpu.VMEM((CH, D), jnp.float32),
            pltpu.SemaphoreType.DMA,
        ],
    )
    def k(y_hbm, p0_hbm, p1_hbm, g0_hbm, g1_hbm, out_hbm,
          i0_v, i1_v, g0_v, g1_v, re_v, ro_v, out_v, sem):
        wid = lax.axis_index("s") * 2 + lax.axis_index("c")
        for c in range(TPW // CH):
            tbase = wid * TPW + c * CH
            sl = pl.ds(tbase, CH)
            pltpu.sync_copy(p0_hbm.at[sl], i0_v)
            pltpu.sync_copy(p1_hbm.at[sl], i1_v)
            pltpu.sync_copy(g0_hbm.at[sl], g0_v)
            pltpu.sync_copy(g1_hbm.at[sl], g1_v)
            a = pltpu.async_copy(y_hbm.at[i0_v], re_v, sem)
            b = pltpu.async_copy(y_hbm.at[i1_v], ro_v, sem)
            a.wait()
            b.wait()

            @plsc.parallel_loop(0, CH * (D // 16), unroll=8)
            def _(n):
                t = lax.shift_right_logical(n, 6)
                s = pl.ds((n & (D // 16 - 1)) * 16, 16)
                out_v[t, s] = g0_v[t, :] * re_v[t, s] + g1_v[t, :] * ro_v[t, s]
            pltpu.sync_copy(out_v, out_hbm.at[sl])

    return k(y, pos0, pos1, g0, g1)


def kernel(x, router_w, w1, w2, w3):
    flat = x.reshape(T, D)
    p0, p1, g0, g1, be_act = _router(flat, router_w)
    p0 = p0.reshape(T)
    p1 = p1.reshape(T)
    xg = _dispatch(flat, p0, p1)
    y = _moe(be_act, xg, w1, w3, w2)
    out = _combine(y, p0, p1, g0, g1)
    return out.reshape(x.shape)


# speedup vs baseline: 1.6986x; 1.6986x over previous
import functools

import jax
import jax.numpy as jnp
from jax import lax
from jax.experimental import pallas as pl
from jax.experimental.pallas import tpu as pltpu
from jax.experimental.pallas import tpu_sc as plsc

T = 2048
D = 1024
F = 4096
E = 8
BM = 256
NB = T * 2 // BM + E
NPAD = NB * BM
NW = 32
TPW = T // NW
CH = 32
CHUNK = 256


def _router_body(x_ref, rw_ref, p0_ref, p1_ref, g0_ref, g1_ref, ba_ref):
    x = x_ref[...]
    rw = rw_ref[...]
    logits = lax.dot_general(x, rw, (((1,), (1,)), ((), ())),
                             preferred_element_type=jnp.float32)
    iota = lax.broadcasted_iota(jnp.int32, logits.shape, 1)
    m1 = jnp.max(logits, axis=1, keepdims=True)
    i1 = jnp.min(jnp.where(logits == m1, iota, E), axis=1, keepdims=True)
    oh0 = (iota == i1)
    l2 = jnp.where(oh0, -jnp.inf, logits)
    m2 = jnp.max(l2, axis=1, keepdims=True)
    i2 = jnp.min(jnp.where(l2 == m2, iota, E), axis=1, keepdims=True)
    oh1 = (iota == i2)
    g1 = 1.0 / (1.0 + jnp.exp(m2 - m1))
    g0_ref[...] = jnp.broadcast_to(g1, (T, 16))
    g1_ref[...] = jnp.broadcast_to(1.0 - g1, (T, 16))

    oh0f = oh0.astype(jnp.float32)
    oh1f = oh1.astype(jnp.float32)
    oh = oh0f + oh1f

    r = lax.broadcasted_iota(jnp.int32, (CHUNK, CHUNK), 0)
    c = lax.broadcasted_iota(jnp.int32, (CHUNK, CHUNK), 1)
    tri = (r >= c).astype(jnp.float32)
    nch = T // CHUNK
    incs = []
    for ci in range(nch):
        oh_c = lax.slice(oh, (ci * CHUNK, 0), ((ci + 1) * CHUNK, E))
        incs.append(lax.dot_general(tri, oh_c, (((1,), (0,)), ((), ())),
                                    preferred_element_type=jnp.float32))
    tot = jnp.concatenate(
        [lax.slice(inc, (CHUNK - 1, 0), (CHUNK, E)) for inc in incs], axis=0)
    r8 = lax.broadcasted_iota(jnp.int32, (nch, nch), 0)
    c8 = lax.broadcasted_iota(jnp.int32, (nch, nch), 1)
    strict8 = (r8 < c8).astype(jnp.float32)
    chunk_excl = lax.dot_general(strict8, tot, (((0,), (0,)), ((), ())),
                                 preferred_element_type=jnp.float32)
    counts = jnp.sum(tot, axis=0, keepdims=True)

    nblk = jnp.floor((counts + (BM - 1)) * (1.0 / BM))
    rE = lax.broadcasted_iota(jnp.int32, (E, E), 0)
    cE = lax.broadcasted_iota(jnp.int32, (E, E), 1)
    strictE = (rE < cE).astype(jnp.float32)
    blk_off = lax.dot_general(nblk, strictE, (((1,), (0,)), ((), ())),
                              preferred_element_type=jnp.float32)
    padded_off = blk_off * BM

    for ci in range(nch):
        oh_c = lax.slice(oh, (ci * CHUNK, 0), ((ci + 1) * CHUNK, E))
        oh0_c = lax.slice(oh0f, (ci * CHUNK, 0), ((ci + 1) * CHUNK, E))
        oh1_c = lax.slice(oh1f, (ci * CHUNK, 0), ((ci + 1) * CHUNK, E))
        excl_c = lax.slice(chunk_excl, (ci, 0), (ci + 1, E))
        before = incs[ci] - oh_c + excl_c + padded_off
        p0 = jnp.sum(before * oh0_c, axis=1, keepdims=True)
        p1 = jnp.sum((before + oh0_c) * oh1_c, axis=1, keepdims=True)
        sl = pl.ds(ci * CHUNK, CHUNK)
        p0_ref[sl, :] = p0.astype(jnp.int32)
        p1_ref[sl, :] = p1.astype(jnp.int32)

    blk_offT = lax.dot_general(jnp.eye(E, dtype=jnp.float32), blk_off,
                               (((1,), (1,)), ((), ())),
                               preferred_element_type=jnp.float32)
    biota = lax.broadcasted_iota(jnp.int32, (E, NB), 1).astype(jnp.float32)
    ge = (biota >= blk_offT).astype(jnp.float32)
    be = jnp.sum(ge, axis=0, keepdims=True) - 1.0
    total = jnp.sum(nblk, axis=1, keepdims=True)
    trow = jnp.broadcast_to(total, (1, NB))
    ba_ref[...] = jnp.concatenate([be, trow], axis=0).astype(jnp.int32)


def _router(flat, router_w):
    return pl.pallas_call(
        _router_body,
        out_shape=(jax.ShapeDtypeStruct((T, 1), jnp.int32),
                   jax.ShapeDtypeStruct((T, 1), jnp.int32),
                   jax.ShapeDtypeStruct((T, 16), jnp.float32),
                   jax.ShapeDtypeStruct((T, 16), jnp.float32),
                   jax.ShapeDtypeStruct((2, NB), jnp.int32)),
    )(flat, router_w)


def _sc_mesh():
    return plsc.VectorSubcoreMesh(core_axis_name="c", subcore_axis_name="s")


def _dispatch(flat, pos0, pos1):
    @functools.partial(
        pl.kernel,
        mesh=_sc_mesh(),
        out_type=jax.ShapeDtypeStruct((NPAD, D), jnp.float32),
        scratch_types=[
            pltpu.VMEM((TPW,), jnp.int32),
            pltpu.VMEM((TPW,), jnp.int32),
            pltpu.VMEM((TPW, D), jnp.float32),
            pltpu.SemaphoreType.DMA,
        ],
    )
    def k(flat_hbm, p0_hbm, p1_hbm, xg_hbm, i0_v, i1_v, rows_v, sem):
        wid = lax.axis_index("s") * 2 + lax.axis_index("c")
        base = wid * TPW
        sl = pl.ds(base, TPW)
        pltpu.sync_copy(p0_hbm.at[sl], i0_v)
        pltpu.sync_copy(p1_hbm.at[sl], i1_v)
        pltpu.sync_copy(flat_hbm.at[sl], rows_v)
        a = pltpu.async_copy(rows_v, xg_hbm.at[i0_v], sem)
        b = pltpu.async_copy(rows_v, xg_hbm.at[i1_v], sem)
        a.wait()
        b.wait()

    return k(flat, pos0, pos1)


BF = 1024
NF = F // BF


def _moe_body(ba_ref, xg_ref, w1_ref, w3_ref, w2_ref, y_ref, acc_ref):
    f = pl.program_id(0)
    b = pl.program_id(1)
    sl = pl.ds(b * BM, BM)

    xb = xg_ref[...]
    a = lax.dot_general(xb, w1_ref[0], (((1,), (1,)), ((), ())),
                        preferred_element_type=jnp.float32)
    c = lax.dot_general(xb, w3_ref[0], (((1,), (1,)), ((), ())),
                        preferred_element_type=jnp.float32)
    h = (a * jax.nn.sigmoid(a)) * c
    p = lax.dot_general(h, w2_ref[0], (((1,), (1,)), ((), ())),
                        preferred_element_type=jnp.float32)

    @pl.when(f == 0)
    def _():
        acc_ref[sl, :] = p

    @pl.when(f > 0)
    def _():
        acc_ref[sl, :] += p

    @pl.when(f == NF - 1)
    def _():
        y_ref[...] = acc_ref[sl, :]


def _moe_outer(ba_ref, xg_hbm, w1_hbm, w3_hbm, w2_hbm, y_hbm, acc_ref):
    def inner(xg_ref, w1_ref, w3_ref, w2_ref, y_ref):
        _moe_body(ba_ref, xg_ref, w1_ref, w3_ref, w2_ref, y_ref, acc_ref)

    total_blocks = ba_ref[1, 0]
    look = pl.Buffered(buffer_count=3, use_lookahead=True)
    pipe = pltpu.emit_pipeline(
        inner,
        grid=(NF, total_blocks),
        in_specs=[
            pl.BlockSpec((BM, D), lambda f, b: (b, 0)),
            pl.BlockSpec((1, BF, D), lambda f, b: (ba_ref[0, b], f, 0),
                         pipeline_mode=look),
            pl.BlockSpec((1, BF, D), lambda f, b: (ba_ref[0, b], f, 0),
                         pipeline_mode=look),
            pl.BlockSpec((1, D, BF), lambda f, b: (ba_ref[0, b], 0, f),
                         pipeline_mode=pl.Buffered(buffer_count=2,
                                                   use_lookahead=True)),
        ],
        out_specs=[pl.BlockSpec(
            (BM, D), lambda f, b: (jnp.where(f == NF - 1, b, 0), 0))],
    )
    pipe(xg_hbm, w1_hbm, w3_hbm, w2_hbm, y_hbm)


def _moe(be_act, xg, w1, w3, w2):
    return pl.pallas_call(
        _moe_outer,
        in_specs=[
            pl.BlockSpec(memory_space=pltpu.SMEM),
            pl.BlockSpec(memory_space=pl.ANY),
            pl.BlockSpec(memory_space=pl.ANY),
            pl.BlockSpec(memory_space=pl.ANY),
            pl.BlockSpec(memory_space=pl.ANY),
        ],
        out_specs=pl.BlockSpec(memory_space=pl.ANY),
        scratch_shapes=[pltpu.VMEM((NPAD, D), jnp.float32)],
        out_shape=jax.ShapeDtypeStruct((NPAD, D), jnp.float32),
        compiler_params=pltpu.CompilerParams(
            vmem_limit_bytes=128 * 1024 * 1024),
    )(be_act, xg, w1, w3, w2)


def _combine(y, pos0, pos1, g0, g1):
    @functools.partial(
        pl.kernel,
        mesh=_sc_mesh(),
        out_type=jax.ShapeDtypeStruct((T, D), jnp.float32),
        scratch_types=[
            pltpu.VMEM((CH,), jnp.int32),
            pltpu.VMEM((CH,), jnp.int32),
            pltpu.VMEM((CH, 16), jnp.float32),
            pltpu.VMEM((CH, 16), jnp.float32),
            pltpu.VMEM((CH, D), jnp.float32),
            pltpu.VMEM((CH, D), jnp.float32),
            pltpu.VMEM((CH, D), jnp.float32),
            pltpu.SemaphoreType.DMA,
        ],
    )
    def k(y_hbm, p0_hbm, p1_hbm, g0_hbm, g1_hbm, out_hbm,
          i0_v, i1_v, g0_v, g1_v, re_v, ro_v, out_v, sem):
        wid = lax.axis_index("s") * 2 + lax.axis_index("c")
        for c in range(TPW // CH):
            tbase = wid * TPW + c * CH
            sl = pl.ds(tbase, CH)
            pltpu.sync_copy(p0_hbm.at[sl], i0_v)
            pltpu.sync_copy(p1_hbm.at[sl], i1_v)
            pltpu.sync_copy(g0_hbm.at[sl], g0_v)
            pltpu.sync_copy(g1_hbm.at[sl], g1_v)
            a = pltpu.async_copy(y_hbm.at[i0_v], re_v, sem)
            b = pltpu.async_copy(y_hbm.at[i1_v], ro_v, sem)
            a.wait()
            b.wait()

            @plsc.parallel_loop(0, CH * (D // 16), unroll=8)
            def _(n):
                t = lax.shift_right_logical(n, 6)
                s = pl.ds((n & (D // 16 - 1)) * 16, 16)
                out_v[t, s] = g0_v[t, :] * re_v[t, s] + g1_v[t, :] * ro_v[t, s]
            pltpu.sync_copy(out_v, out_hbm.at[sl])

    return k(y, pos0, pos1, g0, g1)


def kernel(x, router_w, w1, w2, w3):
    flat = x.reshape(T, D)
    p0, p1, g0, g1, be_act = _router(flat, router_w)
    p0 = p0.reshape(T)
    p1 = p1.reshape(T)
    xg = _dispatch(flat, p0, p1)
    y = _moe(be_act, xg, w1, w3, w2)
    out = _combine(y, p0, p1, g0, g1)
    return out.reshape(x.shape)


# speedup vs baseline: 1.7258x; 1.0160x over previous
import functools

import jax
import jax.numpy as jnp
from jax import lax
from jax.experimental import pallas as pl
from jax.experimental.pallas import tpu as pltpu
from jax.experimental.pallas import tpu_sc as plsc

T = 2048
D = 1024
F = 4096
E = 8
BM = 256
NB = T * 2 // BM + E
NPAD = NB * BM
NW = 32
TPW = T // NW
CH = 16
CHUNK = 256


def _router_body(x_ref, rw_ref, p0_ref, p1_ref, g0_ref, g1_ref, ba_ref):
    x = x_ref[...]
    rw = rw_ref[...]
    logits = lax.dot_general(x, rw, (((1,), (1,)), ((), ())),
                             preferred_element_type=jnp.float32)
    iota = lax.broadcasted_iota(jnp.int32, logits.shape, 1)
    m1 = jnp.max(logits, axis=1, keepdims=True)
    i1 = jnp.min(jnp.where(logits == m1, iota, E), axis=1, keepdims=True)
    oh0 = (iota == i1)
    l2 = jnp.where(oh0, -jnp.inf, logits)
    m2 = jnp.max(l2, axis=1, keepdims=True)
    i2 = jnp.min(jnp.where(l2 == m2, iota, E), axis=1, keepdims=True)
    oh1 = (iota == i2)
    g1 = 1.0 / (1.0 + jnp.exp(m2 - m1))
    g0_ref[...] = jnp.broadcast_to(g1, (T, 16))
    g1_ref[...] = jnp.broadcast_to(1.0 - g1, (T, 16))

    oh0f = oh0.astype(jnp.float32)
    oh1f = oh1.astype(jnp.float32)
    oh = oh0f + oh1f

    r = lax.broadcasted_iota(jnp.int32, (CHUNK, CHUNK), 0)
    c = lax.broadcasted_iota(jnp.int32, (CHUNK, CHUNK), 1)
    tri = (r >= c).astype(jnp.float32)
    nch = T // CHUNK
    incs = []
    for ci in range(nch):
        oh_c = lax.slice(oh, (ci * CHUNK, 0), ((ci + 1) * CHUNK, E))
        incs.append(lax.dot_general(tri, oh_c, (((1,), (0,)), ((), ())),
                                    preferred_element_type=jnp.float32))
    tot = jnp.concatenate(
        [lax.slice(inc, (CHUNK - 1, 0), (CHUNK, E)) for inc in incs], axis=0)
    r8 = lax.broadcasted_iota(jnp.int32, (nch, nch), 0)
    c8 = lax.broadcasted_iota(jnp.int32, (nch, nch), 1)
    strict8 = (r8 < c8).astype(jnp.float32)
    chunk_excl = lax.dot_general(strict8, tot, (((0,), (0,)), ((), ())),
                                 preferred_element_type=jnp.float32)
    counts = jnp.sum(tot, axis=0, keepdims=True)

    nblk = jnp.floor((counts + (BM - 1)) * (1.0 / BM))
    rE = lax.broadcasted_iota(jnp.int32, (E, E), 0)
    cE = lax.broadcasted_iota(jnp.int32, (E, E), 1)
    strictE = (rE < cE).astype(jnp.float32)
    blk_off = lax.dot_general(nblk, strictE, (((1,), (0,)), ((), ())),
                              preferred_element_type=jnp.float32)
    padded_off = blk_off * BM

    for ci in range(nch):
        oh_c = lax.slice(oh, (ci * CHUNK, 0), ((ci + 1) * CHUNK, E))
        oh0_c = lax.slice(oh0f, (ci * CHUNK, 0), ((ci + 1) * CHUNK, E))
        oh1_c = lax.slice(oh1f, (ci * CHUNK, 0), ((ci + 1) * CHUNK, E))
        excl_c = lax.slice(chunk_excl, (ci, 0), (ci + 1, E))
        before = incs[ci] - oh_c + excl_c + padded_off
        p0 = jnp.sum(before * oh0_c, axis=1, keepdims=True)
        p1 = jnp.sum((before + oh0_c) * oh1_c, axis=1, keepdims=True)
        sl = pl.ds(ci * CHUNK, CHUNK)
        p0_ref[sl, :] = p0.astype(jnp.int32)
        p1_ref[sl, :] = p1.astype(jnp.int32)

    blk_offT = lax.dot_general(jnp.eye(E, dtype=jnp.float32), blk_off,
                               (((1,), (1,)), ((), ())),
                               preferred_element_type=jnp.float32)
    biota = lax.broadcasted_iota(jnp.int32, (E, NB), 1).astype(jnp.float32)
    ge = (biota >= blk_offT).astype(jnp.float32)
    be = jnp.sum(ge, axis=0, keepdims=True) - 1.0
    total = jnp.sum(nblk, axis=1, keepdims=True)
    trow = jnp.broadcast_to(total, (1, NB))
    ba_ref[...] = jnp.concatenate([be, trow], axis=0).astype(jnp.int32)


def _router(flat, router_w):
    return pl.pallas_call(
        _router_body,
        out_shape=(jax.ShapeDtypeStruct((T, 1), jnp.int32),
                   jax.ShapeDtypeStruct((T, 1), jnp.int32),
                   jax.ShapeDtypeStruct((T, 16), jnp.float32),
                   jax.ShapeDtypeStruct((T, 16), jnp.float32),
                   jax.ShapeDtypeStruct((2, NB), jnp.int32)),
    )(flat, router_w)


def _sc_mesh():
    return plsc.VectorSubcoreMesh(core_axis_name="c", subcore_axis_name="s")


def _dispatch(flat, pos0, pos1):
    @functools.partial(
        pl.kernel,
        mesh=_sc_mesh(),
        out_type=jax.ShapeDtypeStruct((NPAD, D), jnp.float32),
        scratch_types=[
            pltpu.VMEM((TPW,), jnp.int32),
            pltpu.VMEM((TPW,), jnp.int32),
            pltpu.VMEM((TPW, D), jnp.float32),
            pltpu.SemaphoreType.DMA,
        ],
    )
    def k(flat_hbm, p0_hbm, p1_hbm, xg_hbm, i0_v, i1_v, rows_v, sem):
        wid = lax.axis_index("s") * 2 + lax.axis_index("c")
        base = wid * TPW
        sl = pl.ds(base, TPW)
        pltpu.sync_copy(p0_hbm.at[sl], i0_v)
        pltpu.sync_copy(p1_hbm.at[sl], i1_v)
        pltpu.sync_copy(flat_hbm.at[sl], rows_v)
        a = pltpu.async_copy(rows_v, xg_hbm.at[i0_v], sem)
        b = pltpu.async_copy(rows_v, xg_hbm.at[i1_v], sem)
        a.wait()
        b.wait()

    return k(flat, pos0, pos1)


BF = 1024
NF = F // BF


def _moe_body(ba_ref, xg_ref, w1_ref, w3_ref, w2_ref, y_ref, acc_ref):
    f = pl.program_id(0)
    b = pl.program_id(1)
    sl = pl.ds(b * BM, BM)

    xb = xg_ref[...]
    a = lax.dot_general(xb, w1_ref[0], (((1,), (1,)), ((), ())),
                        preferred_element_type=jnp.float32)
    c = lax.dot_general(xb, w3_ref[0], (((1,), (1,)), ((), ())),
                        preferred_element_type=jnp.float32)
    h = (a * jax.nn.sigmoid(a)) * c
    p = lax.dot_general(h, w2_ref[0], (((1,), (1,)), ((), ())),
                        preferred_element_type=jnp.float32)

    @pl.when(f == 0)
    def _():
        acc_ref[sl, :] = p

    @pl.when(f > 0)
    def _():
        acc_ref[sl, :] += p

    @pl.when(f == NF - 1)
    def _():
        y_ref[...] = acc_ref[sl, :]


def _moe_outer(ba_ref, xg_hbm, w1_hbm, w3_hbm, w2_hbm, y_hbm, acc_ref):
    def inner(xg_ref, w1_ref, w3_ref, w2_ref, y_ref):
        _moe_body(ba_ref, xg_ref, w1_ref, w3_ref, w2_ref, y_ref, acc_ref)

    total_blocks = ba_ref[1, 0]
    look = pl.Buffered(buffer_count=3, use_lookahead=True)
    pipe = pltpu.emit_pipeline(
        inner,
        grid=(NF, total_blocks),
        in_specs=[
            pl.BlockSpec((BM, D), lambda f, b: (b, 0)),
            pl.BlockSpec((1, BF, D), lambda f, b: (ba_ref[0, b], f, 0),
                         pipeline_mode=look),
            pl.BlockSpec((1, BF, D), lambda f, b: (ba_ref[0, b], f, 0),
                         pipeline_mode=look),
            pl.BlockSpec((1, D, BF), lambda f, b: (ba_ref[0, b], 0, f),
                         pipeline_mode=pl.Buffered(buffer_count=2,
                                                   use_lookahead=True)),
        ],
        out_specs=[pl.BlockSpec(
            (BM, D), lambda f, b: (jnp.where(f == NF - 1, b, 0), 0))],
    )
    pipe(xg_hbm, w1_hbm, w3_hbm, w2_hbm, y_hbm)


def _moe(be_act, xg, w1, w3, w2):
    return pl.pallas_call(
        _moe_outer,
        in_specs=[
            pl.BlockSpec(memory_space=pltpu.SMEM),
            pl.BlockSpec(memory_space=pl.ANY),
            pl.BlockSpec(memory_space=pl.ANY),
            pl.BlockSpec(memory_space=pl.ANY),
            pl.BlockSpec(memory_space=pl.ANY),
        ],
        out_specs=pl.BlockSpec(memory_space=pl.ANY),
        scratch_shapes=[pltpu.VMEM((NPAD, D), jnp.float32)],
        out_shape=jax.ShapeDtypeStruct((NPAD, D), jnp.float32),
        compiler_params=pltpu.CompilerParams(
            vmem_limit_bytes=128 * 1024 * 1024),
    )(be_act, xg, w1, w3, w2)


def _combine(y, pos0, pos1, g0, g1):
    @functools.partial(
        pl.kernel,
        mesh=_sc_mesh(),
        out_type=jax.ShapeDtypeStruct((T, D), jnp.float32),
        scratch_types=[
            pltpu.VMEM((TPW,), jnp.int32),
            pltpu.VMEM((TPW,), jnp.int32),
            pltpu.VMEM((TPW, 16), jnp.float32),
            pltpu.VMEM((TPW, 16), jnp.float32),
            pltpu.VMEM((2, CH, D), jnp.float32),
            pltpu.VMEM((2, CH, D), jnp.float32),
            pltpu.VMEM((CH, D), jnp.float32),
            pltpu.SemaphoreType.DMA,
            pltpu.SemaphoreType.DMA,
        ],
    )
    def k(y_hbm, p0_hbm, p1_hbm, g0_hbm, g1_hbm, out_hbm,
          i0_v, i1_v, g0_v, g1_v, re_v, ro_v, out_v, sem0, sem1):
        wid = lax.axis_index("s") * 2 + lax.axis_index("c")
        base = wid * TPW
        wsl = pl.ds(base, TPW)
        pltpu.sync_copy(p0_hbm.at[wsl], i0_v)
        pltpu.sync_copy(p1_hbm.at[wsl], i1_v)
        pltpu.sync_copy(g0_hbm.at[wsl], g0_v)
        pltpu.sync_copy(g1_hbm.at[wsl], g1_v)
        nch = TPW // CH
        sems = (sem0, sem1)

        def issue(c):
            slot = c % 2
            csl = pl.ds(c * CH, CH)
            a = pltpu.async_copy(y_hbm.at[i0_v.at[csl]], re_v.at[slot],
                                 sems[slot])
            b = pltpu.async_copy(y_hbm.at[i1_v.at[csl]], ro_v.at[slot],
                                 sems[slot])
            return a, b

        pending = {0: issue(0)}
        for c in range(nch):
            slot = c % 2
            a, b = pending.pop(c)
            a.wait()
            b.wait()
            if c + 1 < nch:
                pending[c + 1] = issue(c + 1)

            @plsc.parallel_loop(0, CH * (D // 16), unroll=8)
            def _(n):
                t = lax.shift_right_logical(n, 6)
                s = pl.ds((n & (D // 16 - 1)) * 16, 16)
                out_v[t, s] = (g0_v[c * CH + t, :] * re_v[slot, t, s]
                               + g1_v[c * CH + t, :] * ro_v[slot, t, s])
            pltpu.sync_copy(out_v, out_hbm.at[pl.ds(base + c * CH, CH)])

    return k(y, pos0, pos1, g0, g1)


def kernel(x, router_w, w1, w2, w3):
    flat = x.reshape(T, D)
    p0, p1, g0, g1, be_act = _router(flat, router_w)
    p0 = p0.reshape(T)
    p1 = p1.reshape(T)
    xg = _dispatch(flat, p0, p1)
    y = _moe(be_act, xg, w1, w3, w2)
    out = _combine(y, p0, p1, g0, g1)
    return out.reshape(x.shape)
